# Initial kernel scaffold; baseline (speedup 1.0000x reference)
import jax
import jax.numpy as jnp
from jax.experimental import pallas as pl


def kernel(node_feature, edge_index, W, b):
    raise NotImplementedError("write your pallas kernel here")


# speedup vs baseline: 30.3591x; 30.3591x over previous
import jax
import jax.numpy as jnp
from jax import lax
from jax.experimental import pallas as pl
from jax.experimental.pallas import tpu as pltpu
from jax.experimental.pallas import tpu_sc as plsc

N = 10000
E = 320000
D = 128
G = 3

NC = 2
NS = 16
NW = NC * NS

EDGES_PER_TILE = E // NW
CHUNK = 80
NCHUNK = EDGES_PER_TILE // CHUNK
NP = 10112
ROWS_PER_TILE = NP // NS
NB = -(-ROWS_PER_TILE // CHUNK)
NB_LAST = ROWS_PER_TILE - (NB - 1) * CHUNK

PBLK = 1264


def _prep_body(nf_ref, wt_ref, b_ref, out_ref):
    x = nf_ref[...]
    s = jnp.dot(x, wt_ref[...], preferred_element_type=jnp.float32)
    s = s + b_ref[...]
    s = s - jnp.max(s, axis=1, keepdims=True)
    e = jnp.exp(s)
    coeff = e / jnp.sum(e, axis=1, keepdims=True)
    out_ref[0] = coeff
    for g in range(G):
        out_ref[g + 1] = x * coeff[:, g:g + 1]


def _prep(node_feature, W, b):
    wt = jnp.zeros((D, D), jnp.float32).at[:, :G].set(W.T)
    bp = jnp.full((1, D), -1e30, jnp.float32).at[0, :G].set(b)
    nf_p = jnp.zeros((NP, D), jnp.float32).at[:N].set(node_feature)
    return pl.pallas_call(
        _prep_body,
        grid=(NP // PBLK,),
        in_specs=[
            pl.BlockSpec((PBLK, D), lambda i: (i, 0)),
            pl.BlockSpec((D, D), lambda i: (0, 0)),
            pl.BlockSpec((1, D), lambda i: (0, 0)),
        ],
        out_specs=pl.BlockSpec((G + 1, PBLK, D), lambda i: (0, i, 0)),
        out_shape=jax.ShapeDtypeStruct((G + 1, NP, D), jnp.float32),
    )(nf_p, wt, bp)


def _edge_body(prep_h, src_h, dst_h, zrow_h, ones_h,
               acc_out,
               src_v, dst_v, rows_v, ones_v,
               acc_sh, sem):
    c = lax.axis_index("c")
    s = lax.axis_index("s")
    r0 = s * ROWS_PER_TILE
    wid = c * NS + s
    ebase = wid * EDGES_PER_TILE
    pltpu.sync_copy(ones_h, ones_v)

    for g in range(G + 1):
        pltpu.sync_copy(zrow_h, rows_v)
        for k in range(NB):
            sz = CHUNK if k < NB - 1 else NB_LAST
            pltpu.sync_copy(rows_v.at[pl.ds(0, sz)],
                            acc_sh.at[pl.ds(r0 + k * CHUNK, sz)])
        plsc.subcore_barrier()

        def chunk_body(k, carry, g=g):
            base = ebase + k * CHUNK
            pltpu.sync_copy(dst_h.at[pl.ds(base, CHUNK)], dst_v)
            if g < G:
                pltpu.sync_copy(src_h.at[pl.ds(base, CHUNK)], src_v)
                pltpu.async_copy(
                    prep_h.at[g + 1].at[src_v], rows_v, sem).wait()
                pltpu.sync_copy(rows_v, acc_sh.at[dst_v], add=True)
            else:
                pltpu.sync_copy(ones_v, acc_sh.at[dst_v], add=True)
            return carry

        lax.fori_loop(0, NCHUNK, chunk_body, 0)
        plsc.subcore_barrier()

        for k in range(NB):
            sz = CHUNK if k < NB - 1 else NB_LAST
            sl = pl.ds(r0 + k * CHUNK, sz)
            pltpu.sync_copy(acc_sh.at[sl], rows_v.at[pl.ds(0, sz)])
            pltpu.sync_copy(rows_v.at[pl.ds(0, sz)], acc_out.at[g, c, sl])


def _edge_pass(prep, src, dst):
    mesh = plsc.VectorSubcoreMesh(core_axis_name="c", subcore_axis_name="s")
    zrow = jnp.zeros((CHUNK, D), jnp.float32)
    ones = jnp.zeros((CHUNK, D), jnp.float32).at[:, 0].set(1.0)
    fn = pl.kernel(
        _edge_body,
        out_type=jax.ShapeDtypeStruct((G + 1, NC, NP, D), jnp.float32),
        mesh=mesh,
        scratch_types=[
            pltpu.VMEM((CHUNK,), jnp.int32),
            pltpu.VMEM((CHUNK,), jnp.int32),
            pltpu.VMEM((CHUNK, D), jnp.float32),
            pltpu.VMEM((CHUNK, D), jnp.float32),
            pltpu.VMEM_SHARED((NP, D), jnp.float32),
            pltpu.SemaphoreType.DMA,
        ],
    )
    return fn(prep, src, dst, zrow, ones)


BLK = 1000


def _combine_body(acc_ref, coeff_ref, out_ref):
    dg = acc_ref[G, 0, :, 0:1] + acc_ref[G, 1, :, 0:1]
    a = jnp.zeros((BLK, D), jnp.float32)
    for g in range(G):
        sg = acc_ref[g, 0] + acc_ref[g, 1]
        a = a + sg * coeff_ref[:, g:g + 1]
    out_ref[...] = a * (1.0 / jnp.maximum(dg, 1.0))


def _combine(acc_part, coeff):
    return pl.pallas_call(
        _combine_body,
        grid=(N // BLK,),
        in_specs=[
            pl.BlockSpec((G + 1, NC, BLK, D), lambda i: (0, 0, i, 0)),
            pl.BlockSpec((BLK, D), lambda i: (i, 0)),
        ],
        out_specs=pl.BlockSpec((BLK, D), lambda i: (i, 0)),
        out_shape=jax.ShapeDtypeStruct((N, D), jnp.float32),
    )(acc_part[:, :, :N], coeff)


def kernel(node_feature, edge_index, W, b):
    src = edge_index[0].astype(jnp.int32)
    dst = edge_index[1].astype(jnp.int32)
    prep = _prep(node_feature, W, b)
    acc_part = _edge_pass(prep, src, dst)
    return _combine(acc_part, prep[0, :N])


# speedup vs baseline: 45.7056x; 1.5055x over previous
import jax
import jax.numpy as jnp
from jax import lax
from jax.experimental import pallas as pl
from jax.experimental.pallas import tpu as pltpu
from jax.experimental.pallas import tpu_sc as plsc

N = 10000
E = 320000
D = 128
G = 3

NC = 2
NS = 16
NW = NC * NS

EDGES_PER_TILE = E // NW
CHUNK = 80
NCHUNK = EDGES_PER_TILE // CHUNK
NP = 10112
ROWS_PER_TILE = NP // NS
NB = -(-ROWS_PER_TILE // CHUNK)
NB_LAST = ROWS_PER_TILE - (NB - 1) * CHUNK

PBLK = 1264


def _prep_body(nf_ref, wt_ref, b_ref, out_ref):
    x = nf_ref[...]
    s = jnp.dot(x, wt_ref[...], preferred_element_type=jnp.float32)
    s = s + b_ref[...]
    s = s - jnp.max(s, axis=1, keepdims=True)
    e = jnp.exp(s)
    coeff = e / jnp.sum(e, axis=1, keepdims=True)
    out_ref[0] = coeff
    for g in range(G):
        out_ref[g + 1] = x * coeff[:, g:g + 1]


def _prep(node_feature, W, b):
    wt = jnp.zeros((D, D), jnp.float32).at[:, :G].set(W.T)
    bp = jnp.full((1, D), -1e30, jnp.float32).at[0, :G].set(b)
    nf_p = jnp.zeros((NP, D), jnp.float32).at[:N].set(node_feature)
    return pl.pallas_call(
        _prep_body,
        grid=(NP // PBLK,),
        in_specs=[
            pl.BlockSpec((PBLK, D), lambda i: (i, 0)),
            pl.BlockSpec((D, D), lambda i: (0, 0)),
            pl.BlockSpec((1, D), lambda i: (0, 0)),
        ],
        out_specs=pl.BlockSpec((G + 1, PBLK, D), lambda i: (0, i, 0)),
        out_shape=jax.ShapeDtypeStruct((G + 1, NP, D), jnp.float32),
    )(nf_p, wt, bp)


def _edge_body(prep_h, src_h, dst_h, zrow_h, ones_h,
               acc_out,
               src_a, src_b, dst_v, rows_v, rows_b, ones_v,
               acc_sh, sem, sem_b):
    c = lax.axis_index("c")
    s = lax.axis_index("s")
    r0 = s * ROWS_PER_TILE
    wid = c * NS + s
    ebase = wid * EDGES_PER_TILE
    pltpu.sync_copy(ones_h, ones_v)

    for g in range(G + 1):
        pltpu.sync_copy(zrow_h, rows_v)
        for k in range(NB):
            sz = CHUNK if k < NB - 1 else NB_LAST
            pltpu.sync_copy(rows_v.at[pl.ds(0, sz)],
                            acc_sh.at[pl.ds(r0 + k * CHUNK, sz)])
        plsc.subcore_barrier()

        if g < G:
            tab = prep_h.at[g + 1]
            pltpu.sync_copy(src_h.at[pl.ds(ebase, CHUNK)], src_a)
            pltpu.async_copy(tab.at[src_a], rows_v, sem)

            def chunk_body(j, carry, tab=tab):
                base = ebase + 2 * j * CHUNK
                pltpu.sync_copy(src_h.at[pl.ds(base + CHUNK, CHUNK)], src_b)
                pltpu.async_copy(tab.at[src_b], rows_b, sem_b)
                pltpu.sync_copy(dst_h.at[pl.ds(base, CHUNK)], dst_v)
                pltpu.make_async_copy(tab.at[src_a], rows_v, sem).wait()
                pltpu.sync_copy(rows_v, acc_sh.at[dst_v], add=True)
                pltpu.sync_copy(src_h.at[pl.ds(base + 2 * CHUNK, CHUNK)],
                                src_a)
                pltpu.async_copy(tab.at[src_a], rows_v, sem)
                pltpu.sync_copy(dst_h.at[pl.ds(base + CHUNK, CHUNK)], dst_v)
                pltpu.make_async_copy(tab.at[src_b], rows_b, sem_b).wait()
                pltpu.sync_copy(rows_b, acc_sh.at[dst_v], add=True)
                return carry

            lax.fori_loop(0, (NCHUNK - 1) // 2, chunk_body, 0)
            base = ebase + (NCHUNK - 1) * CHUNK
            pltpu.sync_copy(dst_h.at[pl.ds(base, CHUNK)], dst_v)
            pltpu.make_async_copy(tab.at[src_a], rows_v, sem).wait()
            pltpu.sync_copy(rows_v, acc_sh.at[dst_v], add=True)
        else:
            def deg_body(k, carry):
                base = ebase + k * CHUNK
                pltpu.sync_copy(dst_h.at[pl.ds(base, CHUNK)], dst_v)
                pltpu.sync_copy(ones_v, acc_sh.at[dst_v], add=True)
                return carry

            lax.fori_loop(0, NCHUNK, deg_body, 0)
        plsc.subcore_barrier()

        for k in range(NB):
            sz = CHUNK if k < NB - 1 else NB_LAST
            sl = pl.ds(r0 + k * CHUNK, sz)
            pltpu.sync_copy(acc_sh.at[sl], rows_v.at[pl.ds(0, sz)])
            pltpu.sync_copy(rows_v.at[pl.ds(0, sz)], acc_out.at[g, c, sl])


def _edge_pass(prep, src, dst):
    mesh = plsc.VectorSubcoreMesh(core_axis_name="c", subcore_axis_name="s")
    zrow = jnp.zeros((CHUNK, D), jnp.float32)
    ones = jnp.zeros((CHUNK, D), jnp.float32).at[:, 0].set(1.0)
    fn = pl.kernel(
        _edge_body,
        out_type=jax.ShapeDtypeStruct((G + 1, NC, NP, D), jnp.float32),
        mesh=mesh,
        scratch_types=[
            pltpu.VMEM((CHUNK,), jnp.int32),
            pltpu.VMEM((CHUNK,), jnp.int32),
            pltpu.VMEM((CHUNK,), jnp.int32),
            pltpu.VMEM((CHUNK, D), jnp.float32),
            pltpu.VMEM((CHUNK, D), jnp.float32),
            pltpu.VMEM((CHUNK, D), jnp.float32),
            pltpu.VMEM_SHARED((NP, D), jnp.float32),
            pltpu.SemaphoreType.DMA,
            pltpu.SemaphoreType.DMA,
        ],
    )
    return fn(prep, src, dst, zrow, ones)


BLK = 1000


def _combine_body(acc_ref, coeff_ref, out_ref):
    dg = acc_ref[G, 0, :, 0:1] + acc_ref[G, 1, :, 0:1]
    a = jnp.zeros((BLK, D), jnp.float32)
    for g in range(G):
        sg = acc_ref[g, 0] + acc_ref[g, 1]
        a = a + sg * coeff_ref[:, g:g + 1]
    out_ref[...] = a * (1.0 / jnp.maximum(dg, 1.0))


def _combine(acc_part, coeff):
    return pl.pallas_call(
        _combine_body,
        grid=(N // BLK,),
        in_specs=[
            pl.BlockSpec((G + 1, NC, BLK, D), lambda i: (0, 0, i, 0)),
            pl.BlockSpec((BLK, D), lambda i: (i, 0)),
        ],
        out_specs=pl.BlockSpec((BLK, D), lambda i: (i, 0)),
        out_shape=jax.ShapeDtypeStruct((N, D), jnp.float32),
    )(acc_part[:, :, :N], coeff)


def kernel(node_feature, edge_index, W, b):
    src = edge_index[0].astype(jnp.int32)
    dst = edge_index[1].astype(jnp.int32)
    prep = _prep(node_feature, W, b)
    acc_part = _edge_pass(prep, src, dst)
    return _combine(acc_part, prep[0, :N])
